# Initial kernel scaffold; baseline (speedup 1.0000x reference)
import jax
import jax.numpy as jnp
from jax.experimental import pallas as pl


def kernel(x, edge_index, weight0, weight1, bias):
    raise NotImplementedError("write your pallas kernel here")


# speedup vs baseline: 151.9632x; 151.9632x over previous
import functools

import jax
import jax.numpy as jnp
from jax import lax
from jax.experimental import pallas as pl
from jax.experimental.pallas import tpu as pltpu
from jax.experimental.pallas import tpu_sc as plsc

N_NODES = 100000
M_EDGES = 6400000
ROW_W = 128
N_ROWS = M_EDGES // ROW_W
NC, NS = 2, 16
NW = NC * NS
N_PAD = 100352
TILE_SLICE = N_PAD // NS
R = 8
N_GROUPS = N_ROWS // R
BASE_G = N_GROUPS // NW
EXTRA_G = N_GROUPS - BASE_G * NW

_MESH = plsc.VectorSubcoreMesh(core_axis_name="c", subcore_axis_name="s")

_LAYER_SCRATCH = (
    pltpu.VMEM_SHARED((N_PAD,), jnp.float32),
    pltpu.VMEM_SHARED((N_PAD,), jnp.float32),
    pltpu.VMEM((TILE_SLICE,), jnp.float32),
    pltpu.VMEM((TILE_SLICE,), jnp.float32),
    pltpu.VMEM((16,), jnp.float32),
    pltpu.VMEM((R, ROW_W), jnp.int32),
    pltpu.VMEM((R, ROW_W), jnp.int32),
    pltpu.VMEM((R, ROW_W), jnp.float32),
    pltpu.SemaphoreType.DMA,
    pltpu.SemaphoreType.DMA,
)


def _edge_phase(src_hbm, dst_hbm, x_sp, acc_sp, src_buf, dst_buf, vals,
                gsem, ssem, w):
    ngroups = BASE_G + jnp.where(w < EXTRA_G, 1, 0)
    gbase = BASE_G * w + jnp.minimum(w, EXTRA_G)

    @pl.loop(0, ngroups)
    def _chunk(i):
        row0 = (gbase + i) * R
        pltpu.sync_copy(src_hbm.at[pl.ds(row0, R)], src_buf)
        pltpu.sync_copy(dst_hbm.at[pl.ds(row0, R)], dst_buf)
        hs = [
            pltpu.async_copy(x_sp.at[src_buf.at[j]], vals.at[j], gsem)
            for j in range(R)
        ]
        for h in hs:
            h.wait()
        hs = [
            pltpu.async_copy(vals.at[j], acc_sp.at[dst_buf.at[j]], ssem,
                             add=True)
            for j in range(R)
        ]
        for h in hs:
            h.wait()


def _layer_body(first, *refs):
    if first:
        (src_hbm, dst_hbm, xin_hbm, out0_hbm, out1_hbm,
         x_sp, acc_sp, stage_a, stage_b, wbuf,
         src_buf, dst_buf, vals, gsem, ssem) = refs
    else:
        (src_hbm, dst_hbm, p0_hbm, p1_hbm, w_hbm, out0_hbm, out1_hbm,
         x_sp, acc_sp, stage_a, stage_b, wbuf,
         src_buf, dst_buf, vals, gsem, ssem) = refs
    c = lax.axis_index("c")
    s = lax.axis_index("s")
    w = c * NS + s
    sl = pl.ds(s * TILE_SLICE, TILE_SLICE)

    if first:
        pltpu.sync_copy(xin_hbm.at[sl], stage_a)
    else:
        pltpu.sync_copy(p0_hbm.at[sl], stage_a)
        pltpu.sync_copy(p1_hbm.at[sl], stage_b)
        pltpu.sync_copy(w_hbm, wbuf)
        wv = wbuf[...]

        @pl.loop(0, TILE_SLICE // 16)
        def _combine(i):
            ix = pl.ds(i * 16, 16)
            stage_a[ix] = jnp.maximum((stage_a[ix] + stage_b[ix]) * wv, 0.0)

    pltpu.sync_copy(stage_a, x_sp.at[sl])

    @pl.loop(0, TILE_SLICE // 16)
    def _zero(i):
        stage_b[pl.ds(i * 16, 16)] = jnp.zeros((16,), jnp.float32)

    pltpu.sync_copy(stage_b, acc_sp.at[sl])
    plsc.subcore_barrier()

    _edge_phase(src_hbm, dst_hbm, x_sp, acc_sp, src_buf, dst_buf, vals,
                gsem, ssem, w)

    plsc.subcore_barrier()

    @pl.when(c == 0)
    def _w0():
        pltpu.sync_copy(acc_sp.at[sl], out0_hbm.at[sl])

    @pl.when(c == 1)
    def _w1():
        pltpu.sync_copy(acc_sp.at[sl], out1_hbm.at[sl])


_FINAL_CH = N_PAD // NW


def _final_body(q0_hbm, q1_hbm, w_hbm, b_hbm, out_hbm, b0, b1, wbuf, bbuf):
    c = lax.axis_index("c")
    s = lax.axis_index("s")
    w = c * NS + s
    sl = pl.ds(w * _FINAL_CH, _FINAL_CH)
    pltpu.sync_copy(q0_hbm.at[sl], b0)
    pltpu.sync_copy(q1_hbm.at[sl], b1)
    pltpu.sync_copy(w_hbm, wbuf)
    pltpu.sync_copy(b_hbm, bbuf)
    wv = wbuf[...]
    bv = bbuf[...]

    @pl.loop(0, _FINAL_CH // 16)
    def _ew(i):
        ix = pl.ds(i * 16, 16)
        z = jnp.maximum((b0[ix] + b1[ix]) * wv, 0.0) + bv
        b0[ix] = 1.0 / (1.0 + jnp.exp(-z))

    pltpu.sync_copy(b0, out_hbm.at[sl])


_PARTIALS = (jax.ShapeDtypeStruct((N_PAD,), jnp.float32),
             jax.ShapeDtypeStruct((N_PAD,), jnp.float32))

_layer1 = pl.kernel(
    functools.partial(_layer_body, True),
    out_type=_PARTIALS,
    mesh=_MESH,
    scratch_types=list(_LAYER_SCRATCH),
)

_layer2 = pl.kernel(
    functools.partial(_layer_body, False),
    out_type=_PARTIALS,
    mesh=_MESH,
    scratch_types=list(_LAYER_SCRATCH),
)

_final = pl.kernel(
    _final_body,
    out_type=jax.ShapeDtypeStruct((N_PAD,), jnp.float32),
    mesh=_MESH,
    scratch_types=[
        pltpu.VMEM((_FINAL_CH,), jnp.float32),
        pltpu.VMEM((_FINAL_CH,), jnp.float32),
        pltpu.VMEM((16,), jnp.float32),
        pltpu.VMEM((16,), jnp.float32),
    ],
)


def kernel(x, edge_index, weight0, weight1, bias):
    src = edge_index[0].astype(jnp.int32).reshape(N_ROWS, ROW_W)
    dst = edge_index[1].astype(jnp.int32).reshape(N_ROWS, ROW_W)
    xp = jnp.pad(jnp.squeeze(x, 1), (0, N_PAD - N_NODES))
    w0 = jnp.full((16,), weight0[0, 0], jnp.float32)
    w1 = jnp.full((16,), weight1[0, 0], jnp.float32)
    bv = jnp.full((16,), bias, jnp.float32)
    p0, p1 = _layer1(src, dst, xp)
    q0, q1 = _layer2(src, dst, p0, p1, w0)
    outp = _final(q0, q1, w1, bv)
    return outp[:N_NODES]


# speedup vs baseline: 354.9638x; 2.3359x over previous
import functools

import jax
import jax.numpy as jnp
from jax import lax
from jax.experimental import pallas as pl
from jax.experimental.pallas import tpu as pltpu
from jax.experimental.pallas import tpu_sc as plsc

N_NODES = 100000
M_EDGES = 6400000
ROW_W = 128
N_ROWS = M_EDGES // ROW_W
NC, NS = 2, 16
NW = NC * NS
N_PAD = 100352
TILE_SLICE = N_PAD // NS
R = 16
N_CHUNKS = N_ROWS // R
BASE_C = N_CHUNKS // NW
EXTRA_C = N_CHUNKS - BASE_C * NW
NBUF = 3

_MESH = plsc.VectorSubcoreMesh(core_axis_name="c", subcore_axis_name="s")

_LAYER_SCRATCH = (
    pltpu.VMEM_SHARED((N_PAD,), jnp.float32),
    pltpu.VMEM_SHARED((N_PAD,), jnp.float32),
    pltpu.VMEM((TILE_SLICE,), jnp.float32),
    pltpu.VMEM((TILE_SLICE,), jnp.float32),
    pltpu.VMEM((16,), jnp.float32),
    pltpu.VMEM((NBUF, R, ROW_W), jnp.int32),
    pltpu.VMEM((NBUF, R, ROW_W), jnp.int32),
    pltpu.VMEM((NBUF, R, ROW_W), jnp.float32),
    pltpu.SemaphoreType.DMA,
    pltpu.SemaphoreType.DMA,
    pltpu.SemaphoreType.DMA,
)


def _edge_phase(src_hbm, dst_hbm, x_sp, acc_sp, src_buf, dst_buf, vals,
                isem, gsem, ssem, w):
    nch = BASE_C + jnp.where(w < EXTRA_C, 1, 0)
    cbase = BASE_C * w + jnp.minimum(w, EXTRA_C)

    def idx_load(b, ci):
        row0 = (cbase + ci) * R
        pltpu.async_copy(src_hbm.at[pl.ds(row0, R)], src_buf.at[b], isem)
        pltpu.async_copy(dst_hbm.at[pl.ds(row0, R)], dst_buf.at[b], isem)

    def idx_wait(b):
        pltpu.make_async_copy(src_hbm.at[pl.ds(0, R)], src_buf.at[b],
                              isem).wait()
        pltpu.make_async_copy(dst_hbm.at[pl.ds(0, R)], dst_buf.at[b],
                              isem).wait()

    def drain_scatters():
        for j in range(R):
            pltpu.make_async_copy(vals.at[0].at[j],
                                  acc_sp.at[dst_buf.at[0].at[j]],
                                  ssem).wait()

    idx_load(0, 0)

    @pl.loop(0, nch)
    def _chunk(i):
        for b in range(NBUF):

            @pl.when(i % NBUF == b)
            def _body(b=b):
                @pl.when(i >= 2)
                def _drain_old():
                    drain_scatters()

                idx_wait(b)

                @pl.when(i + 1 < nch)
                def _prefetch():
                    idx_load((b + 1) % NBUF, i + 1)

                hs = [
                    pltpu.async_copy(x_sp.at[src_buf.at[b].at[j]],
                                     vals.at[b].at[j], gsem)
                    for j in range(R)
                ]
                for h in hs:
                    h.wait()
                for j in range(R):
                    pltpu.async_copy(vals.at[b].at[j],
                                     acc_sp.at[dst_buf.at[b].at[j]], ssem,
                                     add=True)

    drain_scatters()
    drain_scatters()


def _layer_body(first, *refs):
    if first:
        (src_hbm, dst_hbm, xin_hbm, out0_hbm, out1_hbm,
         x_sp, acc_sp, stage_a, stage_b, wbuf,
         src_buf, dst_buf, vals, isem, gsem, ssem) = refs
    else:
        (src_hbm, dst_hbm, p0_hbm, p1_hbm, w_hbm, out0_hbm, out1_hbm,
         x_sp, acc_sp, stage_a, stage_b, wbuf,
         src_buf, dst_buf, vals, isem, gsem, ssem) = refs
    c = lax.axis_index("c")
    s = lax.axis_index("s")
    w = c * NS + s
    sl = pl.ds(s * TILE_SLICE, TILE_SLICE)

    if first:
        pltpu.sync_copy(xin_hbm.at[sl], stage_a)
    else:
        pltpu.sync_copy(p0_hbm.at[sl], stage_a)
        pltpu.sync_copy(p1_hbm.at[sl], stage_b)
        pltpu.sync_copy(w_hbm, wbuf)
        wv = wbuf[...]

        @pl.loop(0, TILE_SLICE // 16)
        def _combine(i):
            ix = pl.ds(i * 16, 16)
            stage_a[ix] = jnp.maximum((stage_a[ix] + stage_b[ix]) * wv, 0.0)

    pltpu.sync_copy(stage_a, x_sp.at[sl])

    @pl.loop(0, TILE_SLICE // 16)
    def _zero(i):
        stage_b[pl.ds(i * 16, 16)] = jnp.zeros((16,), jnp.float32)

    pltpu.sync_copy(stage_b, acc_sp.at[sl])
    plsc.subcore_barrier()

    _edge_phase(src_hbm, dst_hbm, x_sp, acc_sp, src_buf, dst_buf, vals,
                isem, gsem, ssem, w)

    plsc.subcore_barrier()

    @pl.when(c == 0)
    def _w0():
        pltpu.sync_copy(acc_sp.at[sl], out0_hbm.at[sl])

    @pl.when(c == 1)
    def _w1():
        pltpu.sync_copy(acc_sp.at[sl], out1_hbm.at[sl])


_FINAL_CH = N_PAD // NW


def _final_body(q0_hbm, q1_hbm, w_hbm, b_hbm, out_hbm, b0, b1, wbuf, bbuf):
    c = lax.axis_index("c")
    s = lax.axis_index("s")
    w = c * NS + s
    sl = pl.ds(w * _FINAL_CH, _FINAL_CH)
    pltpu.sync_copy(q0_hbm.at[sl], b0)
    pltpu.sync_copy(q1_hbm.at[sl], b1)
    pltpu.sync_copy(w_hbm, wbuf)
    pltpu.sync_copy(b_hbm, bbuf)
    wv = wbuf[...]
    bv = bbuf[...]

    @pl.loop(0, _FINAL_CH // 16)
    def _ew(i):
        ix = pl.ds(i * 16, 16)
        z = jnp.maximum((b0[ix] + b1[ix]) * wv, 0.0) + bv
        b0[ix] = 1.0 / (1.0 + jnp.exp(-z))

    pltpu.sync_copy(b0, out_hbm.at[sl])


_PARTIALS = (jax.ShapeDtypeStruct((N_PAD,), jnp.float32),
             jax.ShapeDtypeStruct((N_PAD,), jnp.float32))

_layer1 = pl.kernel(
    functools.partial(_layer_body, True),
    out_type=_PARTIALS,
    mesh=_MESH,
    scratch_types=list(_LAYER_SCRATCH),
)

_layer2 = pl.kernel(
    functools.partial(_layer_body, False),
    out_type=_PARTIALS,
    mesh=_MESH,
    scratch_types=list(_LAYER_SCRATCH),
)

_final = pl.kernel(
    _final_body,
    out_type=jax.ShapeDtypeStruct((N_PAD,), jnp.float32),
    mesh=_MESH,
    scratch_types=[
        pltpu.VMEM((_FINAL_CH,), jnp.float32),
        pltpu.VMEM((_FINAL_CH,), jnp.float32),
        pltpu.VMEM((16,), jnp.float32),
        pltpu.VMEM((16,), jnp.float32),
    ],
)


def kernel(x, edge_index, weight0, weight1, bias):
    src = edge_index[0].astype(jnp.int32).reshape(N_ROWS, ROW_W)
    dst = edge_index[1].astype(jnp.int32).reshape(N_ROWS, ROW_W)
    xp = jnp.pad(jnp.squeeze(x, 1), (0, N_PAD - N_NODES))
    w0 = jnp.full((16,), weight0[0, 0], jnp.float32)
    w1 = jnp.full((16,), weight1[0, 0], jnp.float32)
    bv = jnp.full((16,), bias, jnp.float32)
    p0, p1 = _layer1(src, dst, xp)
    q0, q1 = _layer2(src, dst, p0, p1, w0)
    outp = _final(q0, q1, w1, bv)
    return outp[:N_NODES]


# speedup vs baseline: 357.2685x; 1.0065x over previous
import functools

import jax
import jax.numpy as jnp
from jax import lax
from jax.experimental import pallas as pl
from jax.experimental.pallas import tpu as pltpu
from jax.experimental.pallas import tpu_sc as plsc

N_NODES = 100000
M_EDGES = 6400000
ROW_W = 128
N_ROWS = M_EDGES // ROW_W
NC, NS = 2, 16
NW = NC * NS
N_PAD = 100352
TILE_SLICE = N_PAD // NS
CH = 2048
N_CHUNKS = M_EDGES // CH
BASE_C = N_CHUNKS // NW
EXTRA_C = N_CHUNKS - BASE_C * NW
NBUF = 3

_MESH = plsc.VectorSubcoreMesh(core_axis_name="c", subcore_axis_name="s")

_LAYER_SCRATCH = (
    pltpu.VMEM_SHARED((N_PAD,), jnp.float32),
    pltpu.VMEM_SHARED((N_PAD,), jnp.float32),
    pltpu.VMEM((TILE_SLICE,), jnp.float32),
    pltpu.VMEM((TILE_SLICE,), jnp.float32),
    pltpu.VMEM((16,), jnp.float32),
    pltpu.VMEM((CH,), jnp.int32),
    pltpu.VMEM((CH,), jnp.int32),
    pltpu.VMEM((CH,), jnp.int32),
    pltpu.VMEM((CH,), jnp.int32),
    pltpu.VMEM((CH,), jnp.int32),
    pltpu.VMEM((CH,), jnp.int32),
    pltpu.VMEM((CH,), jnp.float32),
    pltpu.VMEM((CH,), jnp.float32),
    pltpu.VMEM((CH,), jnp.float32),
    pltpu.SemaphoreType.DMA,
    pltpu.SemaphoreType.DMA,
    pltpu.SemaphoreType.DMA,
)


def _edge_phase(src_hbm, dst_hbm, x_sp, acc_sp, src_bufs, dst_bufs, val_bufs,
                isem, gsem, ssem, w):
    nch = BASE_C + jnp.where(w < EXTRA_C, 1, 0)
    cbase = BASE_C * w + jnp.minimum(w, EXTRA_C)

    def idx_load(b, ci):
        e0 = (cbase + ci) * CH
        pltpu.async_copy(src_hbm.at[pl.ds(e0, CH)], src_bufs[b], isem)
        pltpu.async_copy(dst_hbm.at[pl.ds(e0, CH)], dst_bufs[b], isem)

    def idx_wait(b):
        pltpu.make_async_copy(src_hbm.at[pl.ds(0, CH)], src_bufs[b],
                              isem).wait()
        pltpu.make_async_copy(dst_hbm.at[pl.ds(0, CH)], dst_bufs[b],
                              isem).wait()

    def drain_scatters():
        pltpu.make_async_copy(val_bufs[0], acc_sp.at[dst_bufs[0]],
                              ssem).wait()

    idx_load(0, 0)

    @pl.loop(0, nch)
    def _chunk(i):
        for b in range(NBUF):

            @pl.when(i % NBUF == b)
            def _body(b=b):
                @pl.when(i >= 2)
                def _drain_old():
                    drain_scatters()

                idx_wait(b)

                @pl.when(i + 1 < nch)
                def _prefetch():
                    idx_load((b + 1) % NBUF, i + 1)

                pltpu.async_copy(x_sp.at[src_bufs[b]], val_bufs[b],
                                 gsem).wait()
                pltpu.async_copy(val_bufs[b], acc_sp.at[dst_bufs[b]], ssem,
                                 add=True)

    drain_scatters()
    drain_scatters()


def _layer_body(first, *refs):
    if first:
        (src_hbm, dst_hbm, xin_hbm, out0_hbm, out1_hbm,
         x_sp, acc_sp, stage_a, stage_b, wbuf,
         sb0, sb1, sb2, db0, db1, db2, vb0, vb1, vb2,
         isem, gsem, ssem) = refs
    else:
        (src_hbm, dst_hbm, p0_hbm, p1_hbm, w_hbm, out0_hbm, out1_hbm,
         x_sp, acc_sp, stage_a, stage_b, wbuf,
         sb0, sb1, sb2, db0, db1, db2, vb0, vb1, vb2,
         isem, gsem, ssem) = refs
    src_bufs = (sb0, sb1, sb2)
    dst_bufs = (db0, db1, db2)
    val_bufs = (vb0, vb1, vb2)
    c = lax.axis_index("c")
    s = lax.axis_index("s")
    w = c * NS + s
    sl = pl.ds(s * TILE_SLICE, TILE_SLICE)

    if first:
        pltpu.sync_copy(xin_hbm.at[sl], stage_a)
    else:
        pltpu.sync_copy(p0_hbm.at[sl], stage_a)
        pltpu.sync_copy(p1_hbm.at[sl], stage_b)
        pltpu.sync_copy(w_hbm, wbuf)
        wv = wbuf[...]

        @pl.loop(0, TILE_SLICE // 16)
        def _combine(i):
            ix = pl.ds(i * 16, 16)
            stage_a[ix] = jnp.maximum((stage_a[ix] + stage_b[ix]) * wv, 0.0)

    pltpu.sync_copy(stage_a, x_sp.at[sl])

    @pl.loop(0, TILE_SLICE // 16)
    def _zero(i):
        stage_b[pl.ds(i * 16, 16)] = jnp.zeros((16,), jnp.float32)

    pltpu.sync_copy(stage_b, acc_sp.at[sl])
    plsc.subcore_barrier()

    _edge_phase(src_hbm, dst_hbm, x_sp, acc_sp, src_bufs, dst_bufs, val_bufs,
                isem, gsem, ssem, w)

    plsc.subcore_barrier()

    @pl.when(c == 0)
    def _w0():
        pltpu.sync_copy(acc_sp.at[sl], out0_hbm.at[sl])

    @pl.when(c == 1)
    def _w1():
        pltpu.sync_copy(acc_sp.at[sl], out1_hbm.at[sl])


_FINAL_CH = N_PAD // NW


def _final_body(q0_hbm, q1_hbm, w_hbm, b_hbm, out_hbm, b0, b1, wbuf, bbuf):
    c = lax.axis_index("c")
    s = lax.axis_index("s")
    w = c * NS + s
    sl = pl.ds(w * _FINAL_CH, _FINAL_CH)
    pltpu.sync_copy(q0_hbm.at[sl], b0)
    pltpu.sync_copy(q1_hbm.at[sl], b1)
    pltpu.sync_copy(w_hbm, wbuf)
    pltpu.sync_copy(b_hbm, bbuf)
    wv = wbuf[...]
    bv = bbuf[...]

    @pl.loop(0, _FINAL_CH // 16)
    def _ew(i):
        ix = pl.ds(i * 16, 16)
        z = jnp.maximum((b0[ix] + b1[ix]) * wv, 0.0) + bv
        b0[ix] = 1.0 / (1.0 + jnp.exp(-z))

    pltpu.sync_copy(b0, out_hbm.at[sl])


_PARTIALS = (jax.ShapeDtypeStruct((N_PAD,), jnp.float32),
             jax.ShapeDtypeStruct((N_PAD,), jnp.float32))

_layer1 = pl.kernel(
    functools.partial(_layer_body, True),
    out_type=_PARTIALS,
    mesh=_MESH,
    scratch_types=list(_LAYER_SCRATCH),
)

_layer2 = pl.kernel(
    functools.partial(_layer_body, False),
    out_type=_PARTIALS,
    mesh=_MESH,
    scratch_types=list(_LAYER_SCRATCH),
)

_final = pl.kernel(
    _final_body,
    out_type=jax.ShapeDtypeStruct((N_PAD,), jnp.float32),
    mesh=_MESH,
    scratch_types=[
        pltpu.VMEM((_FINAL_CH,), jnp.float32),
        pltpu.VMEM((_FINAL_CH,), jnp.float32),
        pltpu.VMEM((16,), jnp.float32),
        pltpu.VMEM((16,), jnp.float32),
    ],
)


def kernel(x, edge_index, weight0, weight1, bias):
    src = edge_index[0].astype(jnp.int32)
    dst = edge_index[1].astype(jnp.int32)
    xp = jnp.pad(jnp.squeeze(x, 1), (0, N_PAD - N_NODES))
    w0 = jnp.full((16,), weight0[0, 0], jnp.float32)
    w1 = jnp.full((16,), weight1[0, 0], jnp.float32)
    bv = jnp.full((16,), bias, jnp.float32)
    p0, p1 = _layer1(src, dst, xp)
    q0, q1 = _layer2(src, dst, p0, p1, w0)
    outp = _final(q0, q1, w1, bv)
    return outp[:N_NODES]


# speedup vs baseline: 391.4641x; 1.0957x over previous
import jax
import jax.numpy as jnp
from jax import lax
from jax.experimental import pallas as pl
from jax.experimental.pallas import tpu as pltpu
from jax.experimental.pallas import tpu_sc as plsc

N_NODES = 100000
M_EDGES = 6400000
NC, NS = 2, 16
NW = NC * NS
N_PAD = 100352
TILE_SLICE = N_PAD // NS
CH = 2048
N_CHUNKS = M_EDGES // CH
BASE_C = N_CHUNKS // NW
EXTRA_C = N_CHUNKS - BASE_C * NW
NBUF = 3

_MESH = plsc.VectorSubcoreMesh(core_axis_name="c", subcore_axis_name="s")

_LAYER_SCRATCH = (
    pltpu.VMEM((N_PAD,), jnp.float32),
    pltpu.VMEM_SHARED((N_PAD,), jnp.float32),
    pltpu.VMEM((TILE_SLICE // 2,), jnp.float32),
    pltpu.VMEM((CH,), jnp.int32),
    pltpu.VMEM((CH,), jnp.int32),
    pltpu.VMEM((CH,), jnp.int32),
    pltpu.VMEM((CH,), jnp.int32),
    pltpu.VMEM((CH,), jnp.int32),
    pltpu.VMEM((CH,), jnp.int32),
    pltpu.VMEM((CH,), jnp.float32),
    pltpu.VMEM((CH,), jnp.float32),
    pltpu.VMEM((CH,), jnp.float32),
    pltpu.SemaphoreType.DMA,
    pltpu.SemaphoreType.DMA,
    pltpu.SemaphoreType.DMA,
)


def _layer_body(*refs):
    (src_hbm, dst_hbm, x_hbm, out0_hbm, out1_hbm,
     x_tile, acc_sp, zbuf,
     sb0, sb1, sb2, db0, db1, db2, vb0, vb1, vb2,
     isem, tsem, ssem) = refs
    src_bufs = (sb0, sb1, sb2)
    dst_bufs = (db0, db1, db2)
    val_bufs = (vb0, vb1, vb2)
    c = lax.axis_index("c")
    s = lax.axis_index("s")
    w = c * NS + s

    xh = pltpu.async_copy(x_hbm, x_tile, tsem)

    @pl.loop(0, (TILE_SLICE // 2) // 16)
    def _zero(k):
        zbuf[pl.ds(k * 16, 16)] = jnp.zeros((16,), jnp.float32)

    pltpu.sync_copy(zbuf, acc_sp.at[pl.ds(s * TILE_SLICE, TILE_SLICE // 2)])
    pltpu.sync_copy(
        zbuf, acc_sp.at[pl.ds(s * TILE_SLICE + TILE_SLICE // 2,
                              TILE_SLICE // 2)])
    xh.wait()
    plsc.subcore_barrier()

    nch = BASE_C + jnp.where(w < EXTRA_C, 1, 0)
    cbase = BASE_C * w + jnp.minimum(w, EXTRA_C)

    def idx_load(b, ci):
        e0 = (cbase + ci) * CH
        pltpu.async_copy(src_hbm.at[pl.ds(e0, CH)], src_bufs[b], isem)
        pltpu.async_copy(dst_hbm.at[pl.ds(e0, CH)], dst_bufs[b], isem)

    def idx_wait(b):
        pltpu.make_async_copy(src_hbm.at[pl.ds(0, CH)], src_bufs[b],
                              isem).wait()
        pltpu.make_async_copy(dst_hbm.at[pl.ds(0, CH)], dst_bufs[b],
                              isem).wait()

    def drain_scatter():
        pltpu.make_async_copy(val_bufs[0], acc_sp.at[dst_bufs[0]],
                              ssem).wait()

    idx_load(0, 0)

    @pl.loop(0, nch)
    def _chunk(i):
        for b in range(NBUF):

            @pl.when(i % NBUF == b)
            def _body(b=b):
                idx_wait(b)

                @pl.when(i >= 2)
                def _drain_old():
                    drain_scatter()

                @pl.when(i + 1 < nch)
                def _prefetch():
                    idx_load((b + 1) % NBUF, i + 1)

                sbuf, vbuf = src_bufs[b], val_bufs[b]

                @pl.loop(0, CH // 16, unroll=8)
                def _gather(k):
                    ix = pl.ds(k * 16, 16)
                    vbuf[ix] = plsc.load_gather(x_tile, [sbuf[ix]])

                pltpu.async_copy(val_bufs[b], acc_sp.at[dst_bufs[b]], ssem,
                                 add=True)

    drain_scatter()
    drain_scatter()
    plsc.subcore_barrier()

    sl = pl.ds(s * TILE_SLICE, TILE_SLICE)

    @pl.when(c == 0)
    def _w0():
        pltpu.sync_copy(acc_sp.at[sl], out0_hbm.at[sl])

    @pl.when(c == 1)
    def _w1():
        pltpu.sync_copy(acc_sp.at[sl], out1_hbm.at[sl])


_EW_CH = N_PAD // NW


def _combine_body(p0_hbm, p1_hbm, w_hbm, out_hbm, b0, b1, wbuf):
    c = lax.axis_index("c")
    s = lax.axis_index("s")
    w = c * NS + s
    sl = pl.ds(w * _EW_CH, _EW_CH)
    pltpu.sync_copy(p0_hbm.at[sl], b0)
    pltpu.sync_copy(p1_hbm.at[sl], b1)
    pltpu.sync_copy(w_hbm, wbuf)
    wv = wbuf[...]

    @pl.loop(0, _EW_CH // 16)
    def _ew(i):
        ix = pl.ds(i * 16, 16)
        b0[ix] = jnp.maximum((b0[ix] + b1[ix]) * wv, 0.0)

    pltpu.sync_copy(b0, out_hbm.at[sl])


def _final_body(q0_hbm, q1_hbm, w_hbm, b_hbm, out_hbm, b0, b1, wbuf, bbuf):
    c = lax.axis_index("c")
    s = lax.axis_index("s")
    w = c * NS + s
    sl = pl.ds(w * _EW_CH, _EW_CH)
    pltpu.sync_copy(q0_hbm.at[sl], b0)
    pltpu.sync_copy(q1_hbm.at[sl], b1)
    pltpu.sync_copy(w_hbm, wbuf)
    pltpu.sync_copy(b_hbm, bbuf)
    wv = wbuf[...]
    bv = bbuf[...]

    @pl.loop(0, _EW_CH // 16)
    def _ew(i):
        ix = pl.ds(i * 16, 16)
        z = jnp.maximum((b0[ix] + b1[ix]) * wv, 0.0) + bv
        b0[ix] = 1.0 / (1.0 + jnp.exp(-z))

    pltpu.sync_copy(b0, out_hbm.at[sl])


_PARTIALS = (jax.ShapeDtypeStruct((N_PAD,), jnp.float32),
             jax.ShapeDtypeStruct((N_PAD,), jnp.float32))

_layer = pl.kernel(
    _layer_body,
    out_type=_PARTIALS,
    mesh=_MESH,
    scratch_types=list(_LAYER_SCRATCH),
    compiler_params=pltpu.CompilerParams(needs_layout_passes=False),
)

_combine = pl.kernel(
    _combine_body,
    out_type=jax.ShapeDtypeStruct((N_PAD,), jnp.float32),
    mesh=_MESH,
    scratch_types=[
        pltpu.VMEM((_EW_CH,), jnp.float32),
        pltpu.VMEM((_EW_CH,), jnp.float32),
        pltpu.VMEM((16,), jnp.float32),
    ],
)

_final = pl.kernel(
    _final_body,
    out_type=jax.ShapeDtypeStruct((N_PAD,), jnp.float32),
    mesh=_MESH,
    scratch_types=[
        pltpu.VMEM((_EW_CH,), jnp.float32),
        pltpu.VMEM((_EW_CH,), jnp.float32),
        pltpu.VMEM((16,), jnp.float32),
        pltpu.VMEM((16,), jnp.float32),
    ],
)


def kernel(x, edge_index, weight0, weight1, bias):
    src = edge_index[0].astype(jnp.int32)
    dst = edge_index[1].astype(jnp.int32)
    xp = jnp.pad(jnp.squeeze(x, 1), (0, N_PAD - N_NODES))
    w0 = jnp.full((16,), weight0[0, 0], jnp.float32)
    w1 = jnp.full((16,), weight1[0, 0], jnp.float32)
    bv = jnp.full((16,), bias, jnp.float32)
    p0, p1 = _layer(src, dst, xp)
    x1 = _combine(p0, p1, w0)
    q0, q1 = _layer(src, dst, x1)
    outp = _final(q0, q1, w1, bv)
    return outp[:N_NODES]


# speedup vs baseline: 396.2929x; 1.0123x over previous
import functools

import jax
import jax.numpy as jnp
from jax import lax
from jax.experimental import pallas as pl
from jax.experimental.pallas import tpu as pltpu
from jax.experimental.pallas import tpu_sc as plsc

N_NODES = 100000
M_EDGES = 6400000
NC, NS = 2, 16
NW = NC * NS
N_PAD = 100352
TILE_SLICE = N_PAD // NS
CH = 2048
N_CHUNKS = M_EDGES // CH
BASE_C = N_CHUNKS // NW
EXTRA_C = N_CHUNKS - BASE_C * NW
NBUF = 3

_MESH = plsc.VectorSubcoreMesh(core_axis_name="c", subcore_axis_name="s")

_LAYER_SCRATCH = (
    pltpu.VMEM((N_PAD,), jnp.float32),
    pltpu.VMEM_SHARED((N_PAD,), jnp.float32),
    pltpu.VMEM((TILE_SLICE // 2,), jnp.float32),
    pltpu.VMEM((16,), jnp.float32),
    pltpu.VMEM((CH,), jnp.int32),
    pltpu.VMEM((CH,), jnp.int32),
    pltpu.VMEM((CH,), jnp.int32),
    pltpu.VMEM((CH,), jnp.int32),
    pltpu.VMEM((CH,), jnp.int32),
    pltpu.VMEM((CH,), jnp.int32),
    pltpu.VMEM((CH,), jnp.float32),
    pltpu.VMEM((CH,), jnp.float32),
    pltpu.VMEM((CH,), jnp.float32),
    pltpu.SemaphoreType.DMA,
    pltpu.SemaphoreType.DMA,
    pltpu.SemaphoreType.DMA,
)


def _layer_body(combine, *refs):
    if combine:
        (src_hbm, dst_hbm, p0_hbm, p1_hbm, w_hbm, out0_hbm, out1_hbm,
         x_tile, acc_sp, zbuf, wbuf,
         sb0, sb1, sb2, db0, db1, db2, vb0, vb1, vb2,
         isem, tsem, ssem) = refs
    else:
        (src_hbm, dst_hbm, x_hbm, out0_hbm, out1_hbm,
         x_tile, acc_sp, zbuf, wbuf,
         sb0, sb1, sb2, db0, db1, db2, vb0, vb1, vb2,
         isem, tsem, ssem) = refs
    src_bufs = (sb0, sb1, sb2)
    dst_bufs = (db0, db1, db2)
    val_bufs = (vb0, vb1, vb2)
    c = lax.axis_index("c")
    s = lax.axis_index("s")
    w = c * NS + s

    HALF = TILE_SLICE // 2
    if combine:
        pltpu.sync_copy(w_hbm, wbuf)
        wv = wbuf[...]
        for t, width in ((0, CH), (1, CH), (2, CH), (3, TILE_SLICE - 3 * CH)):
            sl_t = pl.ds(s * TILE_SLICE + t * CH, width)
            pltpu.sync_copy(p0_hbm.at[sl_t], vb0.at[pl.ds(0, width)])
            pltpu.sync_copy(p1_hbm.at[sl_t], vb1.at[pl.ds(0, width)])

            @pl.loop(0, width // 16)
            def _comb(k):
                ix = pl.ds(k * 16, 16)
                vb0[ix] = jnp.maximum((vb0[ix] + vb1[ix]) * wv, 0.0)

            pltpu.sync_copy(vb0.at[pl.ds(0, width)], acc_sp.at[sl_t])
        plsc.subcore_barrier()
        pltpu.sync_copy(acc_sp, x_tile)
        plsc.subcore_barrier()
    else:
        xh = pltpu.async_copy(x_hbm, x_tile.at[pl.ds(0, N_NODES)], tsem)

    @pl.loop(0, HALF // 16)
    def _zero(k):
        zbuf[pl.ds(k * 16, 16)] = jnp.zeros((16,), jnp.float32)

    pltpu.sync_copy(zbuf, acc_sp.at[pl.ds(s * TILE_SLICE, HALF)])
    pltpu.sync_copy(zbuf, acc_sp.at[pl.ds(s * TILE_SLICE + HALF, HALF)])
    if not combine:
        xh.wait()
    plsc.subcore_barrier()

    nch = BASE_C + jnp.where(w < EXTRA_C, 1, 0)
    cbase = BASE_C * w + jnp.minimum(w, EXTRA_C)

    def idx_load(b, ci):
        e0 = (cbase + ci) * CH
        pltpu.async_copy(src_hbm.at[pl.ds(e0, CH)], src_bufs[b], isem)
        pltpu.async_copy(dst_hbm.at[pl.ds(e0, CH)], dst_bufs[b], isem)

    def idx_wait(b):
        pltpu.make_async_copy(src_hbm.at[pl.ds(0, CH)], src_bufs[b],
                              isem).wait()
        pltpu.make_async_copy(dst_hbm.at[pl.ds(0, CH)], dst_bufs[b],
                              isem).wait()

    def drain_scatter():
        pltpu.make_async_copy(val_bufs[0], acc_sp.at[dst_bufs[0]],
                              ssem).wait()

    idx_load(0, 0)

    @pl.loop(0, nch)
    def _chunk(i):
        for b in range(NBUF):

            @pl.when(i % NBUF == b)
            def _body(b=b):
                idx_wait(b)

                @pl.when(i >= 2)
                def _drain_old():
                    drain_scatter()

                @pl.when(i + 1 < nch)
                def _prefetch():
                    idx_load((b + 1) % NBUF, i + 1)

                sbuf, vbuf = src_bufs[b], val_bufs[b]

                @pl.loop(0, CH // 16, unroll=8)
                def _gather(k):
                    ix = pl.ds(k * 16, 16)
                    vbuf[ix] = plsc.load_gather(x_tile, [sbuf[ix]])

                pltpu.async_copy(val_bufs[b], acc_sp.at[dst_bufs[b]], ssem,
                                 add=True)

    drain_scatter()
    drain_scatter()
    plsc.subcore_barrier()

    sl = pl.ds(s * TILE_SLICE, TILE_SLICE)

    @pl.when(c == 0)
    def _w0():
        pltpu.sync_copy(acc_sp.at[sl], out0_hbm.at[sl])

    @pl.when(c == 1)
    def _w1():
        pltpu.sync_copy(acc_sp.at[sl], out1_hbm.at[sl])


_EW_CH = N_PAD // NW


def _final_body(q0_hbm, q1_hbm, w_hbm, b_hbm, out_hbm, b0, b1, wbuf, bbuf):
    c = lax.axis_index("c")
    s = lax.axis_index("s")
    w = c * NS + s
    sl = pl.ds(w * _EW_CH, _EW_CH)
    pltpu.sync_copy(q0_hbm.at[sl], b0)
    pltpu.sync_copy(q1_hbm.at[sl], b1)
    pltpu.sync_copy(w_hbm, wbuf)
    pltpu.sync_copy(b_hbm, bbuf)
    wv = wbuf[...]
    bv = bbuf[...]

    @pl.loop(0, _EW_CH // 16)
    def _ew(i):
        ix = pl.ds(i * 16, 16)
        z = jnp.maximum((b0[ix] + b1[ix]) * wv, 0.0) + bv
        b0[ix] = 1.0 / (1.0 + jnp.exp(-z))

    pltpu.sync_copy(b0, out_hbm.at[sl])


_PARTIALS = (jax.ShapeDtypeStruct((N_PAD,), jnp.float32),
             jax.ShapeDtypeStruct((N_PAD,), jnp.float32))

_layer1 = pl.kernel(
    functools.partial(_layer_body, False),
    out_type=_PARTIALS,
    mesh=_MESH,
    scratch_types=list(_LAYER_SCRATCH),
    compiler_params=pltpu.CompilerParams(needs_layout_passes=False),
)

_layer2 = pl.kernel(
    functools.partial(_layer_body, True),
    out_type=_PARTIALS,
    mesh=_MESH,
    scratch_types=list(_LAYER_SCRATCH),
    compiler_params=pltpu.CompilerParams(needs_layout_passes=False),
)

_final = pl.kernel(
    _final_body,
    out_type=jax.ShapeDtypeStruct((N_PAD,), jnp.float32),
    mesh=_MESH,
    scratch_types=[
        pltpu.VMEM((_EW_CH,), jnp.float32),
        pltpu.VMEM((_EW_CH,), jnp.float32),
        pltpu.VMEM((16,), jnp.float32),
        pltpu.VMEM((16,), jnp.float32),
    ],
)


def kernel(x, edge_index, weight0, weight1, bias):
    src = edge_index[0].astype(jnp.int32)
    dst = edge_index[1].astype(jnp.int32)
    xv = jnp.squeeze(x, 1)
    w0 = jnp.full((16,), weight0[0, 0], jnp.float32)
    w1 = jnp.full((16,), weight1[0, 0], jnp.float32)
    bv = jnp.full((16,), bias, jnp.float32)
    p0, p1 = _layer1(src, dst, xv)
    q0, q1 = _layer2(src, dst, p0, p1, w0)
    outp = _final(q0, q1, w1, bv)
    return outp[:N_NODES]


# speedup vs baseline: 396.4253x; 1.0003x over previous
import functools

import jax
import jax.numpy as jnp
from jax import lax
from jax.experimental import pallas as pl
from jax.experimental.pallas import tpu as pltpu
from jax.experimental.pallas import tpu_sc as plsc

N_NODES = 100000
M_EDGES = 6400000
NC, NS = 2, 16
NW = NC * NS
N_PAD = 100352
TILE_SLICE = N_PAD // NS
CH = 2048
N_CHUNKS = M_EDGES // CH
BASE_C = N_CHUNKS // NW
EXTRA_C = N_CHUNKS - BASE_C * NW
NBUF = 3

_MESH = plsc.VectorSubcoreMesh(core_axis_name="c", subcore_axis_name="s")

_LAYER_SCRATCH = (
    pltpu.VMEM((N_PAD,), jnp.float32),
    pltpu.VMEM_SHARED((N_PAD,), jnp.float32),
    pltpu.VMEM((TILE_SLICE // 2,), jnp.float32),
    pltpu.VMEM((16,), jnp.float32),
    pltpu.VMEM((CH,), jnp.int32),
    pltpu.VMEM((CH,), jnp.int32),
    pltpu.VMEM((CH,), jnp.int32),
    pltpu.VMEM((CH,), jnp.int32),
    pltpu.VMEM((CH,), jnp.int32),
    pltpu.VMEM((CH,), jnp.int32),
    pltpu.VMEM((CH,), jnp.float32),
    pltpu.VMEM((CH,), jnp.float32),
    pltpu.VMEM((CH,), jnp.float32),
    pltpu.SemaphoreType.DMA,
    pltpu.SemaphoreType.DMA,
    pltpu.SemaphoreType.DMA,
)


def _layer_body(combine, *refs):
    if combine:
        (src_hbm, dst_hbm, p0_hbm, p1_hbm, w_hbm, out0_hbm, out1_hbm,
         x_tile, acc_sp, zbuf, wbuf,
         sb0, sb1, sb2, db0, db1, db2, vb0, vb1, vb2,
         isem, tsem, ssem) = refs
    else:
        (src_hbm, dst_hbm, x_hbm, out0_hbm, out1_hbm,
         x_tile, acc_sp, zbuf, wbuf,
         sb0, sb1, sb2, db0, db1, db2, vb0, vb1, vb2,
         isem, tsem, ssem) = refs
    src_bufs = (sb0, sb1, sb2)
    dst_bufs = (db0, db1, db2)
    val_bufs = (vb0, vb1, vb2)
    c = lax.axis_index("c")
    s = lax.axis_index("s")
    w = c * NS + s

    HALF = TILE_SLICE // 2
    if combine:
        pltpu.sync_copy(w_hbm, wbuf)
        wv = wbuf[...]
        for t, width in ((0, CH), (1, CH), (2, CH), (3, TILE_SLICE - 3 * CH)):
            sl_t = pl.ds(s * TILE_SLICE + t * CH, width)
            pltpu.sync_copy(p0_hbm.at[sl_t], vb0.at[pl.ds(0, width)])
            pltpu.sync_copy(p1_hbm.at[sl_t], vb1.at[pl.ds(0, width)])

            @pl.loop(0, width // 16)
            def _comb(k):
                ix = pl.ds(k * 16, 16)
                vb0[ix] = jnp.maximum((vb0[ix] + vb1[ix]) * wv, 0.0)

            pltpu.sync_copy(vb0.at[pl.ds(0, width)], acc_sp.at[sl_t])
        plsc.subcore_barrier()
        pltpu.sync_copy(acc_sp, x_tile)
        plsc.subcore_barrier()
    else:
        xh = pltpu.async_copy(x_hbm, x_tile.at[pl.ds(0, N_NODES)], tsem)

    @pl.loop(0, HALF // 16)
    def _zero(k):
        zbuf[pl.ds(k * 16, 16)] = jnp.zeros((16,), jnp.float32)

    pltpu.sync_copy(zbuf, acc_sp.at[pl.ds(s * TILE_SLICE, HALF)])
    pltpu.sync_copy(zbuf, acc_sp.at[pl.ds(s * TILE_SLICE + HALF, HALF)])
    if not combine:
        xh.wait()
    plsc.subcore_barrier()

    nch = BASE_C + jnp.where(w < EXTRA_C, 1, 0)
    cbase = BASE_C * w + jnp.minimum(w, EXTRA_C)

    def idx_load(b, ci):
        e0 = (cbase + ci) * CH
        pltpu.async_copy(src_hbm.at[pl.ds(e0, CH)], src_bufs[b], isem)
        pltpu.async_copy(dst_hbm.at[pl.ds(e0, CH)], dst_bufs[b], isem)

    def idx_wait(b):
        pltpu.make_async_copy(src_hbm.at[pl.ds(0, CH)], src_bufs[b],
                              isem).wait()
        pltpu.make_async_copy(dst_hbm.at[pl.ds(0, CH)], dst_bufs[b],
                              isem).wait()

    def drain_scatter():
        pltpu.make_async_copy(val_bufs[0], acc_sp.at[dst_bufs[0]],
                              ssem).wait()

    idx_load(0, 0)

    @pl.loop(0, nch)
    def _chunk(i):
        for b in range(NBUF):

            @pl.when(i % NBUF == b)
            def _body(b=b):
                idx_wait(b)

                @pl.when(i >= 2)
                def _drain_old():
                    drain_scatter()

                @pl.when(i + 1 < nch)
                def _prefetch():
                    idx_load((b + 1) % NBUF, i + 1)

                sbuf, vbuf = src_bufs[b], val_bufs[b]

                @pl.loop(0, CH // 16, unroll=8)
                def _gather(k):
                    ix = pl.ds(k * 16, 16)
                    vbuf[ix] = plsc.load_gather(x_tile, [sbuf[ix]])

                pltpu.async_copy(val_bufs[b], acc_sp.at[dst_bufs[b]], ssem,
                                 add=True)

    drain_scatter()
    drain_scatter()
    plsc.subcore_barrier()

    sl = pl.ds(s * TILE_SLICE, TILE_SLICE)

    @pl.when(c == 0)
    def _w0():
        pltpu.sync_copy(acc_sp.at[sl], out0_hbm.at[sl])

    @pl.when(c == 1)
    def _w1():
        pltpu.sync_copy(acc_sp.at[sl], out1_hbm.at[sl])


_EW_CH = N_PAD // NW


def _final_body(q0_hbm, q1_hbm, w_hbm, b_hbm, out_hbm, b0, b1, wbuf, bbuf):
    c = lax.axis_index("c")
    s = lax.axis_index("s")
    w = c * NS + s
    sl = pl.ds(w * _EW_CH, _EW_CH)
    pltpu.sync_copy(q0_hbm.at[sl], b0)
    pltpu.sync_copy(q1_hbm.at[sl], b1)
    pltpu.sync_copy(w_hbm, wbuf)
    pltpu.sync_copy(b_hbm, bbuf)
    wv = wbuf[...]
    bv = bbuf[...]

    @pl.loop(0, _EW_CH // 16)
    def _ew(i):
        ix = pl.ds(i * 16, 16)
        z = jnp.maximum((b0[ix] + b1[ix]) * wv, 0.0) + bv
        b0[ix] = 1.0 / (1.0 + jnp.exp(-z))

    pltpu.sync_copy(b0, out_hbm.at[sl])


_PARTIALS = (jax.ShapeDtypeStruct((N_PAD,), jnp.float32),
             jax.ShapeDtypeStruct((N_PAD,), jnp.float32))

_LAYER_PARAMS = pltpu.CompilerParams(needs_layout_passes=False,
                                     skip_device_barrier=True)

_layer1 = pl.kernel(
    functools.partial(_layer_body, False),
    out_type=_PARTIALS,
    mesh=_MESH,
    scratch_types=list(_LAYER_SCRATCH),
    compiler_params=_LAYER_PARAMS,
)

_layer2 = pl.kernel(
    functools.partial(_layer_body, True),
    out_type=_PARTIALS,
    mesh=_MESH,
    scratch_types=list(_LAYER_SCRATCH),
    compiler_params=_LAYER_PARAMS,
)

_final = pl.kernel(
    _final_body,
    out_type=jax.ShapeDtypeStruct((N_PAD,), jnp.float32),
    mesh=_MESH,
    scratch_types=[
        pltpu.VMEM((_EW_CH,), jnp.float32),
        pltpu.VMEM((_EW_CH,), jnp.float32),
        pltpu.VMEM((16,), jnp.float32),
        pltpu.VMEM((16,), jnp.float32),
    ],
    compiler_params=pltpu.CompilerParams(skip_device_barrier=True),
)


def kernel(x, edge_index, weight0, weight1, bias):
    src = edge_index[0].astype(jnp.int32)
    dst = edge_index[1].astype(jnp.int32)
    xv = jnp.squeeze(x, 1)
    w0 = jnp.full((16,), weight0[0, 0], jnp.float32)
    w1 = jnp.full((16,), weight1[0, 0], jnp.float32)
    bv = jnp.full((16,), bias, jnp.float32)
    p0, p1 = _layer1(src, dst, xv)
    q0, q1 = _layer2(src, dst, p0, p1, w0)
    outp = _final(q0, q1, w1, bv)
    return outp[:N_NODES]


# speedup vs baseline: 396.6572x; 1.0006x over previous
import functools

import jax
import jax.numpy as jnp
from jax import lax
from jax.experimental import pallas as pl
from jax.experimental.pallas import tpu as pltpu
from jax.experimental.pallas import tpu_sc as plsc

N_NODES = 100000
M_EDGES = 6400000
NC, NS = 2, 16
NW = NC * NS
N_PAD = 100352
TILE_SLICE = N_PAD // NS
CH = 2048
N_CHUNKS = M_EDGES // CH
BASE_C = N_CHUNKS // NW
EXTRA_C = N_CHUNKS - BASE_C * NW
NBUF = 3

_MESH = plsc.VectorSubcoreMesh(core_axis_name="c", subcore_axis_name="s")

_LAYER_SCRATCH = (
    pltpu.VMEM((N_PAD,), jnp.float32),
    pltpu.VMEM_SHARED((N_PAD,), jnp.float32),
    pltpu.VMEM((TILE_SLICE // 2,), jnp.float32),
    pltpu.VMEM((16,), jnp.float32),
    pltpu.VMEM((CH,), jnp.int32),
    pltpu.VMEM((CH,), jnp.int32),
    pltpu.VMEM((CH,), jnp.int32),
    pltpu.VMEM((CH,), jnp.int32),
    pltpu.VMEM((CH,), jnp.int32),
    pltpu.VMEM((CH,), jnp.int32),
    pltpu.VMEM((CH,), jnp.float32),
    pltpu.VMEM((CH,), jnp.float32),
    pltpu.VMEM((CH,), jnp.float32),
    pltpu.SemaphoreType.DMA,
    pltpu.SemaphoreType.DMA,
    pltpu.SemaphoreType.DMA,
)


def _layer_body(combine, *refs):
    if combine:
        (src_hbm, dst_hbm, p0_hbm, p1_hbm, w_hbm, out0_hbm, out1_hbm,
         x_tile, acc_sp, zbuf, wbuf,
         sb0, sb1, sb2, db0, db1, db2, vb0, vb1, vb2,
         isem, tsem, ssem) = refs
    else:
        (src_hbm, dst_hbm, x_hbm, out0_hbm, out1_hbm,
         x_tile, acc_sp, zbuf, wbuf,
         sb0, sb1, sb2, db0, db1, db2, vb0, vb1, vb2,
         isem, tsem, ssem) = refs
    src_bufs = (sb0, sb1, sb2)
    dst_bufs = (db0, db1, db2)
    val_bufs = (vb0, vb1, vb2)
    c = lax.axis_index("c")
    s = lax.axis_index("s")
    w = c * NS + s

    HALF = TILE_SLICE // 2
    if combine:
        pltpu.sync_copy(w_hbm, wbuf)
        wv = wbuf[...]
        for t, width in ((0, CH), (1, CH), (2, CH), (3, TILE_SLICE - 3 * CH)):
            sl_t = pl.ds(s * TILE_SLICE + t * CH, width)
            pltpu.sync_copy(p0_hbm.at[sl_t], vb0.at[pl.ds(0, width)])
            pltpu.sync_copy(p1_hbm.at[sl_t], vb1.at[pl.ds(0, width)])

            @pl.loop(0, width // 16)
            def _comb(k):
                ix = pl.ds(k * 16, 16)
                vb0[ix] = jnp.maximum((vb0[ix] + vb1[ix]) * wv, 0.0)

            pltpu.sync_copy(vb0.at[pl.ds(0, width)], acc_sp.at[sl_t])
        plsc.subcore_barrier()
        pltpu.sync_copy(acc_sp, x_tile)
        plsc.subcore_barrier()
    else:
        xh = pltpu.async_copy(x_hbm, x_tile.at[pl.ds(0, N_NODES)], tsem)

    @pl.loop(0, HALF // 16)
    def _zero(k):
        zbuf[pl.ds(k * 16, 16)] = jnp.zeros((16,), jnp.float32)

    pltpu.sync_copy(zbuf, acc_sp.at[pl.ds(s * TILE_SLICE, HALF)])
    pltpu.sync_copy(zbuf, acc_sp.at[pl.ds(s * TILE_SLICE + HALF, HALF)])
    if not combine:
        xh.wait()
    plsc.subcore_barrier()

    nch = BASE_C + jnp.where(w < EXTRA_C, 1, 0)
    cbase = BASE_C * w + jnp.minimum(w, EXTRA_C)

    def idx_load(b, ci):
        e0 = (cbase + ci) * CH
        pltpu.async_copy(src_hbm.at[pl.ds(e0, CH)], src_bufs[b], isem)
        pltpu.async_copy(dst_hbm.at[pl.ds(e0, CH)], dst_bufs[b], isem)

    def idx_wait(b):
        pltpu.make_async_copy(src_hbm.at[pl.ds(0, CH)], src_bufs[b],
                              isem).wait()
        pltpu.make_async_copy(dst_hbm.at[pl.ds(0, CH)], dst_bufs[b],
                              isem).wait()

    def drain_scatter():
        pltpu.make_async_copy(val_bufs[0], acc_sp.at[dst_bufs[0]],
                              ssem).wait()

    idx_load(0, 0)

    @pl.loop(0, nch)
    def _chunk(i):
        for b in range(NBUF):

            @pl.when(i % NBUF == b)
            def _body(b=b):
                idx_wait(b)

                @pl.when(i >= 2)
                def _drain_old():
                    drain_scatter()

                @pl.when(i + 1 < nch)
                def _prefetch():
                    idx_load((b + 1) % NBUF, i + 1)

                sbuf, vbuf = src_bufs[b], val_bufs[b]

                @pl.loop(0, CH // 16, unroll=8)
                def _gather(k):
                    ix = pl.ds(k * 16, 16)
                    vbuf[ix] = plsc.load_gather(x_tile, [sbuf[ix]])

                pltpu.async_copy(val_bufs[b], acc_sp.at[dst_bufs[b]], ssem,
                                 add=True)

    drain_scatter()
    drain_scatter()
    plsc.subcore_barrier()

    sl = pl.ds(s * TILE_SLICE, TILE_SLICE)

    @pl.when(c == 0)
    def _w0():
        pltpu.sync_copy(acc_sp.at[sl], out0_hbm.at[sl])

    @pl.when(c == 1)
    def _w1():
        pltpu.sync_copy(acc_sp.at[sl], out1_hbm.at[sl])


_EW_CH = N_PAD // NW


def _final_body(q0_hbm, q1_hbm, w_hbm, b_hbm, out_hbm, b0, b1, wbuf, bbuf):
    c = lax.axis_index("c")
    s = lax.axis_index("s")
    w = c * NS + s
    sl = pl.ds(w * _EW_CH, _EW_CH)
    pltpu.sync_copy(q0_hbm.at[sl], b0)
    pltpu.sync_copy(q1_hbm.at[sl], b1)
    pltpu.sync_copy(w_hbm, wbuf)
    pltpu.sync_copy(b_hbm, bbuf)
    wv = wbuf[...]
    bv = bbuf[...]

    @pl.loop(0, _EW_CH // 16)
    def _ew(i):
        ix = pl.ds(i * 16, 16)
        z = jnp.maximum((b0[ix] + b1[ix]) * wv, 0.0) + bv
        b0[ix] = 1.0 / (1.0 + jnp.exp(-z))

    pltpu.sync_copy(b0, out_hbm.at[sl])


_PARTIALS = (jax.ShapeDtypeStruct((N_PAD,), jnp.float32),
             jax.ShapeDtypeStruct((N_PAD,), jnp.float32))

_LAYER_PARAMS = pltpu.CompilerParams(needs_layout_passes=False)

_layer1 = pl.kernel(
    functools.partial(_layer_body, False),
    out_type=_PARTIALS,
    mesh=_MESH,
    scratch_types=list(_LAYER_SCRATCH),
    compiler_params=_LAYER_PARAMS,
)

_layer2 = pl.kernel(
    functools.partial(_layer_body, True),
    out_type=_PARTIALS,
    mesh=_MESH,
    scratch_types=list(_LAYER_SCRATCH),
    compiler_params=_LAYER_PARAMS,
)

_final = pl.kernel(
    _final_body,
    out_type=jax.ShapeDtypeStruct((N_PAD,), jnp.float32),
    mesh=_MESH,
    scratch_types=[
        pltpu.VMEM((_EW_CH,), jnp.float32),
        pltpu.VMEM((_EW_CH,), jnp.float32),
        pltpu.VMEM((16,), jnp.float32),
        pltpu.VMEM((16,), jnp.float32),
    ],
)


def kernel(x, edge_index, weight0, weight1, bias):
    src = edge_index[0].astype(jnp.int32)
    dst = edge_index[1].astype(jnp.int32)
    xv = jnp.squeeze(x, 1)
    w0 = jnp.full((16,), weight0[0, 0], jnp.float32)
    w1 = jnp.full((16,), weight1[0, 0], jnp.float32)
    bv = jnp.full((16,), bias, jnp.float32)
    p0, p1 = _layer1(src, dst, xv)
    q0, q1 = _layer2(src, dst, p0, p1, w0)
    outp = _final(q0, q1, w1, bv)
    return outp[:N_NODES]
